# baseline (speedup 1.0000x reference)
import functools

import jax
import jax.numpy as jnp
from jax import lax
from jax.experimental import pallas as pl
from jax.experimental.pallas import tpu as pltpu
from jax.experimental.pallas import tpu_sc as plsc

D_EDGE = 16
N_BINS = 7
W_OUT = D_EDGE + N_BINS
LANES = 16
NC, NS = 2, 16
NW = NC * NS


@functools.lru_cache(maxsize=None)
def _build_sc(E: int, C: int):
    n_chunks = E // C
    n_groups = C // LANES
    per_w = (n_chunks + NW - 1) // NW
    mesh = plsc.VectorSubcoreMesh(core_axis_name="c", subcore_axis_name="s")

    @functools.partial(
        pl.kernel,
        mesh=mesh,
        compiler_params=pltpu.CompilerParams(
            needs_layout_passes=False, use_tc_tiling_on_sc=True
        ),
        out_type=jax.ShapeDtypeStruct((W_OUT, E), jnp.float32),
        scratch_types=[
            pltpu.VMEM((C,), jnp.int32),
            pltpu.VMEM((C,), jnp.int32),
            pltpu.VMEM((N_BINS, C), jnp.float32),
        ],
    )
    def k(s_hbm, r_hbm, o_hbm, s_v, r_v, ob_v):
        wid = lax.axis_index("s") * NC + lax.axis_index("c")

        def chunk_body(j, carry):
            ck = wid + j * NW

            @pl.when(ck < n_chunks)
            def _():
                e0 = ck * C
                pltpu.sync_copy(s_hbm.at[pl.ds(e0, C)], s_v)
                pltpu.sync_copy(r_hbm.at[pl.ds(e0, C)], r_v)

                def group_body(g, gcarry):
                    gb = g * LANES
                    s = s_v[pl.ds(gb, LANES)]
                    r = r_v[pl.ds(gb, LANES)]
                    sep = s - r + 1
                    idx = jnp.clip(sep + 5, 0, 6) + jnp.clip(sep + 10, 0, 1)
                    cls = 6 - idx
                    for c in range(N_BINS):
                        vals = jnp.where(cls == c, 1.0, 0.0).astype(jnp.float32)
                        ob_v[c, pl.ds(gb, LANES)] = vals
                    return gcarry

                lax.fori_loop(0, n_groups, group_body, 0)
                pltpu.sync_copy(
                    ob_v, o_hbm.at[pl.ds(D_EDGE, N_BINS), pl.ds(e0, C)]
                )

            return carry

        lax.fori_loop(0, per_w, chunk_body, 0)

    return k


@functools.lru_cache(maxsize=None)
def _build_tc(E: int, BE: int):
    grid = (E // BE,)

    def body(f_ref, _, o_ref):
        o_ref[...] = f_ref[...]

    return pl.pallas_call(
        body,
        grid=grid,
        in_specs=[
            pl.BlockSpec((D_EDGE, BE), lambda i: (0, i)),
            pl.BlockSpec(memory_space=pl.ANY),
        ],
        out_specs=pl.BlockSpec((D_EDGE, BE), lambda i: (0, i)),
        out_shape=jax.ShapeDtypeStruct((W_OUT, E), jnp.float32),
        input_output_aliases={1: 0},
        compiler_params=pltpu.CompilerParams(
            dimension_semantics=("arbitrary",)
        ),
    )


def kernel(senders, receivers, edge_features):
    E = senders.shape[0]
    C = 3200
    BE = 12800
    assert E % C == 0 and E % BE == 0
    sc_k = _build_sc(E, C)
    tc_k = _build_tc(E, BE)
    oh_t = sc_k(senders, receivers)
    out_t = tc_k(edge_features.T, oh_t)
    return out_t.T


# speedup vs baseline: 1.7825x; 1.7825x over previous
import functools

import jax
import jax.numpy as jnp
from jax import lax
from jax.experimental import pallas as pl
from jax.experimental.pallas import tpu as pltpu
from jax.experimental.pallas import tpu_sc as plsc

D_EDGE = 16
N_BINS = 7
W_OUT = D_EDGE + N_BINS
LANES = 16
NC, NS = 2, 16
NW = NC * NS


@functools.lru_cache(maxsize=None)
def _build(E: int, C: int):
    n_chunks = E // C
    n_groups = C // LANES
    n_rounds = ((n_chunks + NW - 1) // NW + 1) // 2
    mesh = plsc.VectorSubcoreMesh(core_axis_name="c", subcore_axis_name="s")

    @functools.partial(
        pl.kernel,
        mesh=mesh,
        compiler_params=pltpu.CompilerParams(
            needs_layout_passes=False, use_tc_tiling_on_sc=True
        ),
        out_type=jax.ShapeDtypeStruct((W_OUT, E), jnp.float32),
        scratch_types=[
            pltpu.VMEM((2, C), jnp.int32),
            pltpu.VMEM((2, C), jnp.int32),
            pltpu.VMEM((2, N_BINS, C), jnp.float32),
            pltpu.VMEM((2, D_EDGE, C), jnp.float32),
            pltpu.SemaphoreType.DMA,
            pltpu.SemaphoreType.DMA,
            pltpu.SemaphoreType.DMA,
            pltpu.SemaphoreType.DMA,
        ],
    )
    def k(s_hbm, r_hbm, x_hbm, o_hbm, s_v, r_v, ob_v, fb_v, rs0, rs1, ws0, ws1):
        wid = lax.axis_index("s") * NC + lax.axis_index("c")
        rs = (rs0, rs1)
        ws = (ws0, ws1)

        def read_copies(j, b):
            e0 = (wid + j * NW) * C
            return (
                pltpu.make_async_copy(s_hbm.at[pl.ds(e0, C)], s_v.at[b], rs[b]),
                pltpu.make_async_copy(r_hbm.at[pl.ds(e0, C)], r_v.at[b], rs[b]),
                pltpu.make_async_copy(
                    x_hbm.at[:, pl.ds(e0, C)], fb_v.at[b], rs[b]
                ),
            )

        def write_copies(j, b):
            e0 = (wid + j * NW) * C
            return (
                pltpu.make_async_copy(
                    ob_v.at[b],
                    o_hbm.at[pl.ds(D_EDGE, N_BINS), pl.ds(e0, C)],
                    ws[b],
                ),
                pltpu.make_async_copy(
                    fb_v.at[b],
                    o_hbm.at[pl.ds(0, D_EDGE), pl.ds(e0, C)],
                    ws[b],
                ),
            )

        def issue(copies):
            for cp in copies:
                cp.start()

        def drain(copies):
            for cp in copies:
                cp.wait()

        @pl.when(wid < n_chunks)
        def _():
            issue(read_copies(0, 0))

        def round_body(rnd, carry):
            for b in range(2):
                j = rnd * 2 + b
                ck = wid + j * NW

                @pl.when(ck < n_chunks)
                def _(j=j, b=b):
                    @pl.when(ck + NW < n_chunks)
                    def _():
                        if b == 1:
                            drain(write_copies(j - 1, 1 - b))
                        else:

                            @pl.when(rnd >= 1)
                            def _():
                                drain(write_copies(j - 1, 1 - b))

                        issue(read_copies(j + 1, 1 - b))

                    drain(read_copies(j, b))

                    def group_body(g, gcarry):
                        gb = g * LANES
                        s = s_v[b, pl.ds(gb, LANES)]
                        r = r_v[b, pl.ds(gb, LANES)]
                        sep = s - r + 1
                        idx = jnp.clip(sep + 5, 0, 6) + jnp.clip(
                            sep + 10, 0, 1
                        )
                        cls = 6 - idx
                        for c in range(N_BINS):
                            vals = jnp.where(cls == c, 1.0, 0.0).astype(
                                jnp.float32
                            )
                            ob_v[b, c, pl.ds(gb, LANES)] = vals
                        return gcarry

                    lax.fori_loop(0, n_groups, group_body, 0)
                    issue(write_copies(j, b))

            return carry

        lax.fori_loop(0, n_rounds, round_body, 0)

        nj = (n_chunks - wid + NW - 1) // NW
        for b in range(2):

            @pl.when((nj >= 1) & ((nj - 1) % 2 == b))
            def _(b=b):
                drain(write_copies(nj - 1, b))

            @pl.when((nj >= 2) & ((nj - 2) % 2 == b))
            def _(b=b):
                drain(write_copies(nj - 2, b))

    return k


def kernel(senders, receivers, edge_features):
    E = senders.shape[0]
    C = 1280
    assert E % C == 0 and C % 128 == 0
    k = _build(E, C)
    out_t = k(senders, receivers, edge_features.T)
    return out_t.T
